# Initial kernel scaffold; baseline (speedup 1.0000x reference)
import jax
import jax.numpy as jnp
from jax.experimental import pallas as pl


def kernel(label_ids, table):
    raise NotImplementedError("write your pallas kernel here")


# speedup vs baseline: 1.0228x; 1.0228x over previous
import functools

import jax
import jax.numpy as jnp
from jax import lax
from jax.experimental import pallas as pl
from jax.experimental.pallas import tpu as pltpu
from jax.experimental.pallas import tpu_sc as plsc

NUM_LABELS = 1000000
EMBED_DIM = 32
BATCH = 16384
HIST = 50
TOTAL = BATCH * HIST

_NC = 2
_NS = 16
_NW = _NC * _NS
_PER_W = TOTAL // _NW
_CH = 128
_NCHUNK = _PER_W // _CH


def _make_kernel():
  mesh = plsc.VectorSubcoreMesh(core_axis_name="c", subcore_axis_name="s")

  @functools.partial(
      pl.kernel,
      out_type=jax.ShapeDtypeStruct((TOTAL, EMBED_DIM), jnp.float32),
      mesh=mesh,
      compiler_params=pltpu.CompilerParams(use_tc_tiling_on_sc=False),
      scratch_types=[
          pltpu.VMEM((_NCHUNK, _CH), jnp.int32),
          pltpu.VMEM((_CH, EMBED_DIM), jnp.float32),
          pltpu.SemaphoreType.DMA,
      ],
  )
  def gather_kernel(idx_hbm, table_hbm, out_hbm, idx_v, rows_v, sem):
    wid = lax.axis_index("s") * _NC + lax.axis_index("c")
    base = wid * _PER_W
    pltpu.sync_copy(idx_hbm.at[wid], idx_v)

    def body(j, _):
      pltpu.async_copy(table_hbm.at[idx_v.at[j]], rows_v, sem).wait()
      pltpu.sync_copy(rows_v, out_hbm.at[pl.ds(base + j * _CH, _CH)])
      return 0

    lax.fori_loop(0, _NCHUNK, body, 0)

  return gather_kernel


_gather = _make_kernel()


def kernel(label_ids, table):
  idx = label_ids.astype(jnp.int32).reshape(_NW, _NCHUNK, _CH)
  out = _gather(idx, table)
  return out.reshape(BATCH, HIST, EMBED_DIM)


# speedup vs baseline: 1.0791x; 1.0550x over previous
import functools

import jax
import jax.numpy as jnp
from jax import lax
from jax.experimental import pallas as pl
from jax.experimental.pallas import tpu as pltpu
from jax.experimental.pallas import tpu_sc as plsc

NUM_LABELS = 1000000
EMBED_DIM = 32
BATCH = 16384
HIST = 50
TOTAL = BATCH * HIST

_NC = 2
_NS = 16
_NW = _NC * _NS
_PER_W = TOTAL // _NW
_CH = 128
_NCHUNK = _PER_W // _CH


def _make_kernel():
  mesh = plsc.VectorSubcoreMesh(core_axis_name="c", subcore_axis_name="s")

  @functools.partial(
      pl.kernel,
      out_type=jax.ShapeDtypeStruct((TOTAL, EMBED_DIM), jnp.float32),
      mesh=mesh,
      compiler_params=pltpu.CompilerParams(use_tc_tiling_on_sc=False),
      scratch_types=[
          pltpu.VMEM((_NCHUNK, _CH), jnp.int32),
          pltpu.VMEM((_CH, EMBED_DIM), jnp.float32),
          pltpu.VMEM((_CH, EMBED_DIM), jnp.float32),
          pltpu.SemaphoreType.DMA,
          pltpu.SemaphoreType.DMA,
      ],
  )
  def gather_kernel(idx_hbm, table_hbm, out_hbm, idx_v, rows0, rows1, g0, g1):
    wid = lax.axis_index("s") * _NC + lax.axis_index("c")
    base = wid * _PER_W
    pltpu.sync_copy(idx_hbm.at[wid], idx_v)

    pltpu.async_copy(table_hbm.at[idx_v.at[0]], rows0, g0)

    def body(h, _):
      j = h * 2
      up1 = pltpu.async_copy(table_hbm.at[idx_v.at[j + 1]], rows1, g1)
      pltpu.make_async_copy(table_hbm.at[idx_v.at[j]], rows0, g0).wait()
      pltpu.sync_copy(rows0, out_hbm.at[pl.ds(base + j * _CH, _CH)])

      @pl.when(j + 2 < _NCHUNK)
      def _():
        pltpu.async_copy(table_hbm.at[idx_v.at[j + 2]], rows0, g0)

      up1.wait()
      pltpu.sync_copy(rows1, out_hbm.at[pl.ds(base + (j + 1) * _CH, _CH)])
      return 0

    lax.fori_loop(0, _NCHUNK // 2, body, 0)

  return gather_kernel


_gather = _make_kernel()


def kernel(label_ids, table):
  idx = label_ids.astype(jnp.int32).reshape(_NW, _NCHUNK, _CH)
  out = _gather(idx, table)
  return out.reshape(BATCH, HIST, EMBED_DIM)


# speedup vs baseline: 1.5794x; 1.4637x over previous
import functools

import jax
import jax.numpy as jnp
from jax import lax
from jax.experimental import pallas as pl
from jax.experimental.pallas import tpu as pltpu
from jax.experimental.pallas import tpu_sc as plsc

NUM_LABELS = 1000000
EMBED_DIM = 32
BATCH = 16384
HIST = 50

_NC = 2
_NS = 16
_NW = _NC * _NS
_ROWS_W = BATCH // _NW


def _make_kernel():
  mesh = plsc.VectorSubcoreMesh(core_axis_name="c", subcore_axis_name="s")

  @functools.partial(
      pl.kernel,
      out_type=jax.ShapeDtypeStruct((BATCH, HIST, EMBED_DIM), jnp.float32),
      mesh=mesh,
      compiler_params=pltpu.CompilerParams(use_tc_tiling_on_sc=False),
      scratch_types=[
          pltpu.VMEM((_ROWS_W, HIST), jnp.int32),
          pltpu.VMEM((HIST, EMBED_DIM), jnp.float32),
          pltpu.VMEM((HIST, EMBED_DIM), jnp.float32),
          pltpu.SemaphoreType.DMA,
          pltpu.SemaphoreType.DMA,
      ],
  )
  def gather_kernel(idx_hbm, table_hbm, out_hbm, idx_v, rows0, rows1, g0, g1):
    wid = lax.axis_index("s") * _NC + lax.axis_index("c")
    base = wid * _ROWS_W
    pltpu.sync_copy(idx_hbm.at[wid], idx_v)

    pltpu.async_copy(table_hbm.at[idx_v.at[0]], rows0, g0)

    def body(h, _):
      j = h * 2
      up1 = pltpu.async_copy(table_hbm.at[idx_v.at[j + 1]], rows1, g1)
      pltpu.make_async_copy(table_hbm.at[idx_v.at[j]], rows0, g0).wait()
      pltpu.sync_copy(rows0, out_hbm.at[base + j])

      @pl.when(j + 2 < _ROWS_W)
      def _():
        pltpu.async_copy(table_hbm.at[idx_v.at[j + 2]], rows0, g0)

      up1.wait()
      pltpu.sync_copy(rows1, out_hbm.at[base + j + 1])
      return 0

    lax.fori_loop(0, _ROWS_W // 2, body, 0)

  return gather_kernel


_gather = _make_kernel()


def kernel(label_ids, table):
  idx = label_ids.astype(jnp.int32).reshape(_NW, _ROWS_W, HIST)
  return _gather(idx, table)


# speedup vs baseline: 1.7926x; 1.1350x over previous
import functools

import jax
import jax.numpy as jnp
from jax import lax
from jax.experimental import pallas as pl
from jax.experimental.pallas import tpu as pltpu
from jax.experimental.pallas import tpu_sc as plsc

NUM_LABELS = 1000000
EMBED_DIM = 32
BATCH = 16384
HIST = 50

_NC = 2
_NS = 16
_NW = _NC * _NS
_ROWS_W = BATCH // _NW
_RC = 8
_CH = _RC * HIST
_NCHUNK = _ROWS_W // _RC


def _make_kernel():
  mesh = plsc.VectorSubcoreMesh(core_axis_name="c", subcore_axis_name="s")

  @functools.partial(
      pl.kernel,
      out_type=jax.ShapeDtypeStruct((BATCH, HIST, EMBED_DIM), jnp.float32),
      mesh=mesh,
      compiler_params=pltpu.CompilerParams(use_tc_tiling_on_sc=False),
      scratch_types=[
          pltpu.VMEM((_NCHUNK, _CH), jnp.int32),
          pltpu.VMEM((_CH, EMBED_DIM), jnp.float32),
          pltpu.VMEM((_CH, EMBED_DIM), jnp.float32),
          pltpu.SemaphoreType.DMA,
          pltpu.SemaphoreType.DMA,
      ],
  )
  def gather_kernel(idx_hbm, table_hbm, out_hbm, idx_v, rows0, rows1, g0, g1):
    wid = lax.axis_index("s") * _NC + lax.axis_index("c")
    base = wid * _ROWS_W
    pltpu.sync_copy(idx_hbm.at[wid], idx_v)

    pltpu.async_copy(table_hbm.at[idx_v.at[0]], rows0, g0)

    def drain(buf, row0):
      for r in range(_RC):
        pltpu.sync_copy(buf.at[pl.ds(r * HIST, HIST)], out_hbm.at[row0 + r])

    def body(h, _):
      j = h * 2
      up1 = pltpu.async_copy(table_hbm.at[idx_v.at[j + 1]], rows1, g1)
      pltpu.make_async_copy(table_hbm.at[idx_v.at[j]], rows0, g0).wait()
      drain(rows0, base + j * _RC)

      @pl.when(j + 2 < _NCHUNK)
      def _():
        pltpu.async_copy(table_hbm.at[idx_v.at[j + 2]], rows0, g0)

      up1.wait()
      drain(rows1, base + (j + 1) * _RC)
      return 0

    lax.fori_loop(0, _NCHUNK // 2, body, 0)

  return gather_kernel


_gather = _make_kernel()


def kernel(label_ids, table):
  idx = label_ids.astype(jnp.int32).reshape(_NW, _NCHUNK, _CH)
  return _gather(idx, table)
